# baseline (speedup 1.0000x reference)
import functools

import jax
import jax.numpy as jnp
from jax import lax
from jax.experimental import pallas as pl
from jax.experimental.pallas import tpu as pltpu
from jax.experimental.pallas import tpu_sc as plsc

_V = 1000000
_D = 64
_B = 16384

_NC = 2
_NS = 16
_NW = _NC * _NS
_BPW = _B // _NW
_HB = _BPW // 2


def _sc_partials(cw, xw, a2, b2):
    mesh = plsc.VectorSubcoreMesh(core_axis_name="c", subcore_axis_name="s")

    @functools.partial(
        pl.kernel,
        mesh=mesh,
        compiler_params=pltpu.CompilerParams(
            needs_layout_passes=False, use_tc_tiling_on_sc=True),
        out_type=jax.ShapeDtypeStruct((2 * _NW, 16), jnp.float32),
        scratch_types=[
            pltpu.VMEM((_BPW // 32, 32), jnp.int32),
            pltpu.VMEM((_BPW // 32, 32), jnp.int32),
            pltpu.VMEM((32, 1, _D), jnp.float32),
            pltpu.VMEM((32, 1, _D), jnp.float32),
            pltpu.VMEM((16,), jnp.float32),
            pltpu.VMEM((16,), jnp.float32),
            pltpu.SemaphoreType.DMA,
        ],
    )
    def k(cw_hbm, xw_hbm, a2_hbm, b2_hbm, out_hbm,
          idxc, idxx, ga, gb, resm, rese, sem):
        wid = lax.axis_index("s") * _NC + lax.axis_index("c")
        base = wid * _BPW

        for h in range(_BPW // 32):
            pltpu.sync_copy(cw_hbm.at[pl.ds(base + h * 32, 32)], idxc.at[h])
            pltpu.sync_copy(xw_hbm.at[pl.ds(base + h * 32, 32)], idxx.at[h])

        lanes = lax.iota(jnp.int32, 16)
        zero = jnp.zeros((16,), jnp.float32)

        def chunk(h, carry):
            sm, se = carry
            cpa = pltpu.async_copy(a2_hbm.at[idxc.at[h]], ga, sem)
            cpb = pltpu.async_copy(b2_hbm.at[idxx.at[h]], gb, sem)
            cpa.wait()
            cpb.wait()

            def grp_body(g, carry2):
                sm2, se2 = carry2
                dvec = zero
                for j in range(16):
                    r = g * 16 + j
                    lane_j = lanes == j
                    p = zero
                    for kk in range(4):
                        a = ga[r, 0, pl.ds(kk * 16, 16)]
                        b = gb[r, 0, pl.ds(kk * 16, 16)]
                        p = p + a * b
                    dot = jnp.sum(p)
                    dvec = dvec + jnp.where(lane_j, dot, 0.0)
                return sm2 + dvec, se2 + jnp.exp(dvec)

            return lax.fori_loop(0, 2, grp_body, (sm, se))

        sm, se = lax.fori_loop(0, _BPW // 32, chunk, (zero, zero))
        resm[...] = sm
        rese[...] = se
        pltpu.sync_copy(resm, out_hbm.at[wid])
        pltpu.sync_copy(rese, out_hbm.at[_NW + wid])

    return k(cw, xw, a2, b2)


def _tc_finish(p_ref, o_ref):
    x = p_ref[...]
    t = jnp.sum(x[:_NW])
    s = jnp.sum(x[_NW:])
    o_ref[...] = jnp.reshape(jnp.float32(_B) * jnp.log(s) - t, (1, 1))


def kernel(center_word, context_word, center_emb, context_emb):
    cw = center_word.astype(jnp.int32)
    xw = context_word.astype(jnp.int32)
    parts = _sc_partials(cw, xw,
                         center_emb.reshape(_V, 1, _D),
                         context_emb.reshape(_V, 1, _D))
    loss = pl.pallas_call(
        _tc_finish,
        out_shape=jax.ShapeDtypeStruct((1, 1), jnp.float32),
    )(parts)
    return loss[0, 0]


# speedup vs baseline: 1.0236x; 1.0236x over previous
import functools

import jax
import jax.numpy as jnp
from jax import lax
from jax.experimental import pallas as pl
from jax.experimental.pallas import tpu as pltpu
from jax.experimental.pallas import tpu_sc as plsc

_V = 1000000
_D = 64
_B = 16384

_NC = 2
_NS = 16
_NW = _NC * _NS
_BPW = _B // _NW
_HB = _BPW // 2


def _sc_partials(cw, xw, a2, b2):
    mesh = plsc.VectorSubcoreMesh(core_axis_name="c", subcore_axis_name="s")

    @functools.partial(
        pl.kernel,
        mesh=mesh,
        compiler_params=pltpu.CompilerParams(
            needs_layout_passes=False, use_tc_tiling_on_sc=True),
        out_type=jax.ShapeDtypeStruct((2 * _NW, 16), jnp.float32),
        scratch_types=[
            pltpu.VMEM((_BPW // 64, 64), jnp.int32),
            pltpu.VMEM((_BPW // 64, 64), jnp.int32),
            pltpu.VMEM((64, 1, _D), jnp.float32),
            pltpu.VMEM((64, 1, _D), jnp.float32),
            pltpu.VMEM((16,), jnp.float32),
            pltpu.VMEM((16,), jnp.float32),
            pltpu.SemaphoreType.DMA,
        ],
    )
    def k(cw_hbm, xw_hbm, a2_hbm, b2_hbm, out_hbm,
          idxc, idxx, ga, gb, resm, rese, sem):
        wid = lax.axis_index("s") * _NC + lax.axis_index("c")
        base = wid * _BPW

        for h in range(_BPW // 64):
            pltpu.sync_copy(cw_hbm.at[pl.ds(base + h * 64, 64)], idxc.at[h])
            pltpu.sync_copy(xw_hbm.at[pl.ds(base + h * 64, 64)], idxx.at[h])

        lanes = lax.iota(jnp.int32, 16)
        zero = jnp.zeros((16,), jnp.float32)

        def chunk(h, carry):
            sm, se = carry
            cpa = pltpu.async_copy(a2_hbm.at[idxc.at[h]], ga, sem)
            cpb = pltpu.async_copy(b2_hbm.at[idxx.at[h]], gb, sem)
            cpa.wait()
            cpb.wait()

            def grp_body(g, carry2):
                sm2, se2 = carry2
                dvec = zero
                for j in range(16):
                    r = g * 16 + j
                    lane_j = lanes == j
                    p = zero
                    for kk in range(4):
                        a = ga[r, 0, pl.ds(kk * 16, 16)]
                        b = gb[r, 0, pl.ds(kk * 16, 16)]
                        p = p + a * b
                    dot = jnp.sum(p)
                    dvec = dvec + jnp.where(lane_j, dot, 0.0)
                return sm2 + dvec, se2 + jnp.exp(dvec)

            return lax.fori_loop(0, 4, grp_body, (sm, se))

        sm, se = lax.fori_loop(0, _BPW // 64, chunk, (zero, zero))
        resm[...] = sm
        rese[...] = se
        pltpu.sync_copy(resm, out_hbm.at[wid])
        pltpu.sync_copy(rese, out_hbm.at[_NW + wid])

    return k(cw, xw, a2, b2)


def _tc_finish(p_ref, o_ref):
    x = p_ref[...]
    t = jnp.sum(x[:_NW])
    s = jnp.sum(x[_NW:])
    o_ref[...] = jnp.reshape(jnp.float32(_B) * jnp.log(s) - t, (1, 1))


def kernel(center_word, context_word, center_emb, context_emb):
    cw = center_word.astype(jnp.int32)
    xw = context_word.astype(jnp.int32)
    parts = _sc_partials(cw, xw,
                         center_emb.reshape(_V, 1, _D),
                         context_emb.reshape(_V, 1, _D))
    loss = pl.pallas_call(
        _tc_finish,
        out_shape=jax.ShapeDtypeStruct((1, 1), jnp.float32),
    )(parts)
    return loss[0, 0]
